# Initial kernel scaffold; baseline (speedup 1.0000x reference)
import jax
import jax.numpy as jnp
from jax.experimental import pallas as pl


def kernel(sequence, ngrams, word_emb, ngram_emb, W1, b1, W2, b2):
    raise NotImplementedError("write your pallas kernel here")


# speedup vs baseline: 3.9384x; 3.9384x over previous
import functools

import jax
import jax.numpy as jnp
from jax import lax
from jax.experimental import pallas as pl
from jax.experimental.pallas import tpu as pltpu
from jax.experimental.pallas import tpu_sc as plsc

_B = 4096
_S = 200
_D = 64
_H = 256
_C = 10
_NC = 2
_NS = 16
_NW = _NC * _NS
_RPT = _B // _NW
_HS = _S // 2


def _pool_body(seq_hbm, ng_hbm, wtab_hbm, ntab_hbm, out_hbm,
               seq_idx, ng_idx, bufs, acc, sem0, sem1):
    wid = lax.axis_index("c") * _NS + lax.axis_index("s")
    base = wid * _RPT

    pltpu.sync_copy(seq_hbm.at[pl.ds(2 * base, 2 * _RPT)], seq_idx)
    pltpu.sync_copy(ng_hbm.at[pl.ds(2 * base, 2 * _RPT)], ng_idx)

    sems = (sem0, sem1)

    def fire(row, slot):
        j = 2 * row
        b = 4 * slot
        pltpu.async_copy(wtab_hbm.at[seq_idx.at[j]], bufs.at[b + 0], sems[slot])
        pltpu.async_copy(wtab_hbm.at[seq_idx.at[j + 1]], bufs.at[b + 1], sems[slot])
        pltpu.async_copy(ntab_hbm.at[ng_idx.at[j]], bufs.at[b + 2], sems[slot])
        pltpu.async_copy(ntab_hbm.at[ng_idx.at[j + 1]], bufs.at[b + 3], sems[slot])

    def drain(row, slot):
        j = 2 * row
        b = 4 * slot
        pltpu.make_async_copy(wtab_hbm.at[seq_idx.at[j]], bufs.at[b + 0], sems[slot]).wait()
        pltpu.make_async_copy(wtab_hbm.at[seq_idx.at[j + 1]], bufs.at[b + 1], sems[slot]).wait()
        pltpu.make_async_copy(ntab_hbm.at[ng_idx.at[j]], bufs.at[b + 2], sems[slot]).wait()
        pltpu.make_async_copy(ntab_hbm.at[ng_idx.at[j + 1]], bufs.at[b + 3], sems[slot]).wait()

    def reduce_row(row, slot):
        b = 4 * slot

        def body(i, carry):
            out = []
            for k in range(8):
                pair = k // 4
                c = k % 4
                v = (carry[k]
                     + bufs[b + 2 * pair, i, pl.ds(c * 16, 16)]
                     + bufs[b + 2 * pair + 1, i, pl.ds(c * 16, 16)])
                out.append(v)
            return tuple(out)

        zeros = tuple(jnp.zeros((16,), jnp.float32) for _ in range(8))
        sums = lax.fori_loop(0, _HS, body, zeros)
        for k in range(8):
            acc[row, pl.ds(16 * k, 16)] = sums[k]

    fire(0, 0)
    fire(1, 1)

    def outer(rr, carry):
        r = 2 * rr
        for slot in range(2):
            row = r + slot
            drain(row, slot)
            reduce_row(row, slot)
            fire(row + 2, slot)
        return carry

    lax.fori_loop(0, _RPT // 2 - 1, outer, 0)
    for slot in range(2):
        drain(_RPT - 2 + slot, slot)
        reduce_row(_RPT - 2 + slot, slot)

    pltpu.sync_copy(acc, out_hbm.at[pl.ds(base, _RPT)])


@functools.partial(
    pl.kernel,
    mesh=plsc.VectorSubcoreMesh(core_axis_name="c", subcore_axis_name="s"),
    out_type=jax.ShapeDtypeStruct((_B, 2 * _D), jnp.float32),
    scratch_types=[
        pltpu.VMEM((2 * _RPT, _HS), jnp.int32),
        pltpu.VMEM((2 * _RPT, _HS), jnp.int32),
        pltpu.VMEM((8, _HS, _D), jnp.float32),
        pltpu.VMEM((_RPT, 2 * _D), jnp.float32),
        pltpu.SemaphoreType.DMA,
        pltpu.SemaphoreType.DMA,
    ],
    compiler_params=pltpu.CompilerParams(use_tc_tiling_on_sc=False),
)
def _pool(seq_hbm, ng_hbm, wtab_hbm, ntab_hbm, out_hbm,
          seq_idx, ng_idx, bufs, acc, sem0, sem1):
    _pool_body(seq_hbm, ng_hbm, wtab_hbm, ntab_hbm, out_hbm,
               seq_idx, ng_idx, bufs, acc, sem0, sem1)


_BM = 512


def _mlp_body(x_ref, w1_ref, b1_ref, w2_ref, b2_ref, o_ref):
    x = x_ref[...] * (1.0 / _S)
    h = lax.dot_general(x, w1_ref[...], (((1,), (1,)), ((), ())),
                        preferred_element_type=jnp.float32)
    h = jnp.maximum(h + b1_ref[...], 0.0)
    o = lax.dot_general(h, w2_ref[...], (((1,), (1,)), ((), ())),
                        preferred_element_type=jnp.float32)
    o_ref[...] = o + b2_ref[...]


def _mlp(xsum, W1, b1, W2, b2):
    return pl.pallas_call(
        _mlp_body,
        grid=(_B // _BM,),
        in_specs=[
            pl.BlockSpec((_BM, 2 * _D), lambda i: (i, 0)),
            pl.BlockSpec((_H, 2 * _D), lambda i: (0, 0)),
            pl.BlockSpec((1, _H), lambda i: (0, 0)),
            pl.BlockSpec((_C, _H), lambda i: (0, 0)),
            pl.BlockSpec((1, _C), lambda i: (0, 0)),
        ],
        out_specs=pl.BlockSpec((_BM, _C), lambda i: (i, 0)),
        out_shape=jax.ShapeDtypeStruct((_B, _C), jnp.float32),
    )(xsum, W1, b1.reshape(1, _H), W2, b2.reshape(1, _C))


def kernel(sequence, ngrams, word_emb, ngram_emb, W1, b1, W2, b2):
    seq2 = sequence.reshape(2 * _B, _HS).astype(jnp.int32)
    ng2 = ngrams.reshape(2 * _B, _HS).astype(jnp.int32)
    xsum = _pool(seq2, ng2, word_emb, ngram_emb)
    return _mlp(xsum, W1, b1, W2, b2)
